# baseline (speedup 1.0000x reference)
import functools

import jax
import jax.numpy as jnp
from jax import lax
from jax.experimental import pallas as pl
from jax.experimental.pallas import tpu as pltpu
from jax.experimental.pallas import tpu_sc as plsc

B, C, H, W = 32, 256, 64, 64
HW = H * W
N = 20000
L = 16
NW = 32
NCHUNK = N // L
CPW = (NCHUNK + NW - 1) // NW
NB = C // L

_OFFS = [(-1, -1), (-1, 0), (-1, 1), (0, -1), (0, 0), (0, 1),
         (1, -1), (1, 0), (1, 1)]


_DNUMS = lax.GatherDimensionNumbers(
    offset_dims=(), collapsed_slice_dims=(0,), start_index_map=(0,))


def _lane_gather(vec, idx):
    return lax.gather(vec, idx[:, None], _DNUMS, (1,),
                      mode=lax.GatherScatterMode.PROMISE_IN_BOUNDS)


def _floor_i32(x):
    t = x.astype(jnp.int32)
    return jnp.where(t.astype(jnp.float32) > x, t - 1, t)


def _sc_body(tab, dyt, dxt, bids, inds, out,
             bid_v, ind_v, dy_v, dx_v, w_v,
             cen_v, cor_v, blk_v, o1_v, o2_v, sem):
    wid = lax.axis_index("s") * 2 + lax.axis_index("c")

    def chunk_body(t, _):
        ci = wid + NW * t

        @pl.when(ci < NCHUNK)
        def _():
            base = pl.multiple_of(ci * L, L)
            pltpu.sync_copy(bids.at[pl.ds(base, L)], bid_v)
            pltpu.sync_copy(inds.at[pl.ds(base, L)], ind_v)
            bid = bid_v[...]
            ind = ind_v[...]
            brow = bid * HW
            rows_c = brow + ind

            pltpu.async_copy(tab.at[rows_c], cen_v, sem).wait()
            pltpu.sync_copy(cen_v, out.at[0, pl.ds(base, L)])

            pltpu.async_copy(dyt.at[rows_c], dy_v, sem).wait()
            pltpu.async_copy(dxt.at[rows_c], dx_v, sem).wait()

            cy = lax.shift_right_logical(ind, 6)
            cx = jnp.bitwise_and(ind, 63)
            y = cy.astype(jnp.float32) + dy_v[...]
            x = cx.astype(jnp.float32) + dx_v[...]
            x0 = _floor_i32(x)
            y0 = _floor_i32(y)
            x1 = x0 + 1
            y1 = y0 + 1
            wx1 = x - x0.astype(jnp.float32)
            wx0 = 1.0 - wx1
            wy1 = y - y0.astype(jnp.float32)
            wy0 = 1.0 - wy1

            corners = ((y0, x0, wy0 * wx0), (y0, x1, wy0 * wx1),
                       (y1, x0, wy1 * wx0), (y1, x1, wy1 * wx1))
            for k, (yi, xi, wk) in enumerate(corners):
                valid = ((xi >= 0) & (xi <= W - 1)
                         & (yi >= 0) & (yi <= H - 1))
                xc = jnp.minimum(jnp.maximum(xi, 0), W - 1)
                yc = jnp.minimum(jnp.maximum(yi, 0), H - 1)
                rows_k = brow + yc * W + xc
                w_v[k] = jnp.where(valid, wk, 0.0)
                pltpu.async_copy(tab.at[rows_k], cor_v.at[k], sem).wait()

            for m, (dy_o, dx_o) in enumerate(_OFFS):
                ny = jnp.minimum(jnp.maximum(cy + dy_o, 0), H - 1)
                nx = jnp.minimum(jnp.maximum(cx + dx_o, 0), W - 1)
                rows_m = brow + ny * W + nx
                pltpu.async_copy(tab.at[rows_m], blk_v.at[m], sem).wait()

            def det_body(d, _):
                d_idx = jnp.broadcast_to(d, (L,))
                wb = [_lane_gather(w_v[k], d_idx) for k in range(4)]
                for j in range(NB):
                    sl = pl.ds(j * L, L)
                    acc = wb[0] * cor_v[0, d, sl]
                    acc = acc + wb[1] * cor_v[1, d, sl]
                    acc = acc + wb[2] * cor_v[2, d, sl]
                    acc = acc + wb[3] * cor_v[3, d, sl]
                    o1_v[d, sl] = acc
                    mx = blk_v[0, d, sl]
                    for m in range(1, 9):
                        mx = jnp.maximum(mx, blk_v[m, d, sl])
                    o2_v[d, sl] = mx
                return _

            lax.fori_loop(0, L, det_body, None)
            pltpu.sync_copy(o1_v, out.at[1, pl.ds(base, L)])
            pltpu.sync_copy(o2_v, out.at[2, pl.ds(base, L)])

        return _

    lax.fori_loop(0, CPW, chunk_body, None)


@jax.jit
def kernel(params_maps, offset_maps, batch_ids, flat_inds):
    tab = jnp.transpose(params_maps, (0, 2, 3, 1)).reshape(B * HW, C)
    dyt = offset_maps[:, 0, :, :].reshape(B * HW)
    dxt = offset_maps[:, 1, :, :].reshape(B * HW)

    mesh = plsc.VectorSubcoreMesh(core_axis_name="c", subcore_axis_name="s")
    f = pl.kernel(
        _sc_body,
        mesh=mesh,
        out_type=jax.ShapeDtypeStruct((3, N, C), jnp.float32),
        scratch_types=[
            pltpu.VMEM((L,), jnp.int32),
            pltpu.VMEM((L,), jnp.int32),
            pltpu.VMEM((L,), jnp.float32),
            pltpu.VMEM((L,), jnp.float32),
            pltpu.VMEM((4, L), jnp.float32),
            pltpu.VMEM((L, C), jnp.float32),
            pltpu.VMEM((4, L, C), jnp.float32),
            pltpu.VMEM((9, L, C), jnp.float32),
            pltpu.VMEM((L, C), jnp.float32),
            pltpu.VMEM((L, C), jnp.float32),
            pltpu.SemaphoreType.DMA,
        ],
    )
    return f(tab, dyt, dxt, batch_ids, flat_inds)


# speedup vs baseline: 2.2129x; 2.2129x over previous
import jax
import jax.numpy as jnp
from jax import lax
from jax.experimental import pallas as pl
from jax.experimental.pallas import tpu as pltpu
from jax.experimental.pallas import tpu_sc as plsc

B, C, H, W = 32, 256, 64, 64
HW = H * W
N = 20000
L = 16
NW = 32
NCHUNK = N // L
CPW = (NCHUNK + NW - 1) // NW
NB = C // L

_OFFS = [(-1, -1), (-1, 0), (-1, 1), (0, -1), (0, 0), (0, 1),
         (1, -1), (1, 0), (1, 1)]

_DNUMS = lax.GatherDimensionNumbers(
    offset_dims=(), collapsed_slice_dims=(0,), start_index_map=(0,))


def _lane_gather(vec, idx):
    return lax.gather(vec, idx[:, None], _DNUMS, (1,),
                      mode=lax.GatherScatterMode.PROMISE_IN_BOUNDS)


def _floor_i32(x):
    t = x.astype(jnp.int32)
    return jnp.where(t.astype(jnp.float32) > x, t - 1, t)


def _sc_body(tab, dyt, dxt, bids, inds, out,
             bid_v, ind_v, dy_v, dx_v, w_v, idx1_v, idx3_v,
             cen_v, gbuf, o1_v, o2_v,
             sem_o, sem_b, sem_c, sem_w):
    wid = lax.axis_index("s") * 2 + lax.axis_index("c")

    def chunk_body(t, _):
        ci = wid + NW * t

        @pl.when(ci < NCHUNK)
        def _():
            base = pl.multiple_of(ci * L, L)

            @pl.when(t > 0)
            def _():
                pltpu.make_async_copy(
                    cen_v, out.at[0, pl.ds(base, L)], sem_w).wait()
                pltpu.make_async_copy(
                    o1_v, out.at[1, pl.ds(base, L)], sem_w).wait()
                pltpu.make_async_copy(
                    o2_v, out.at[2, pl.ds(base, L)], sem_w).wait()

            d_bi = pltpu.async_copy(bids.at[pl.ds(base, L)], bid_v, sem_o)
            d_ii = pltpu.async_copy(inds.at[pl.ds(base, L)], ind_v, sem_o)
            d_bi.wait()
            d_ii.wait()
            bid = bid_v[...]
            ind = ind_v[...]
            brow = bid * HW
            rows_c = brow + ind

            d_dy = pltpu.async_copy(dyt.at[rows_c], dy_v, sem_o)
            d_dx = pltpu.async_copy(dxt.at[rows_c], dx_v, sem_o)
            d_cen = pltpu.async_copy(tab.at[rows_c], cen_v, sem_b)

            cy = lax.shift_right_logical(ind, 6)
            cx = jnp.bitwise_and(ind, 63)
            rows_blk8 = None
            for m, (dy_o, dx_o) in enumerate(_OFFS):
                ny = jnp.minimum(jnp.maximum(cy + dy_o, 0), H - 1)
                nx = jnp.minimum(jnp.maximum(cx + dx_o, 0), W - 1)
                rows_m = brow + ny * W + nx
                if m < 8:
                    idx1_v[pl.ds(m * L, L)] = rows_m
                else:
                    rows_blk8 = rows_m
            d_g1 = pltpu.async_copy(tab.at[idx1_v],
                                    gbuf.at[pl.ds(0, 8 * L)], sem_b)
            d_g2 = pltpu.async_copy(tab.at[rows_blk8],
                                    gbuf.at[pl.ds(8 * L, L)], sem_b)

            d_dy.wait()
            d_dx.wait()
            y = cy.astype(jnp.float32) + dy_v[...]
            x = cx.astype(jnp.float32) + dx_v[...]
            x0 = _floor_i32(x)
            y0 = _floor_i32(y)
            x1 = x0 + 1
            y1 = y0 + 1
            wx1 = x - x0.astype(jnp.float32)
            wx0 = 1.0 - wx1
            wy1 = y - y0.astype(jnp.float32)
            wy0 = 1.0 - wy1
            corners = ((y0, x0, wy0 * wx0), (y0, x1, wy0 * wx1),
                       (y1, x0, wy1 * wx0), (y1, x1, wy1 * wx1))
            for k, (yi, xi, wk) in enumerate(corners):
                valid = ((xi >= 0) & (xi <= W - 1)
                         & (yi >= 0) & (yi <= H - 1))
                xc = jnp.minimum(jnp.maximum(xi, 0), W - 1)
                yc = jnp.minimum(jnp.maximum(yi, 0), H - 1)
                idx3_v[pl.ds(k * L, L)] = brow + yc * W + xc
                w_v[k] = jnp.where(valid, wk, 0.0)
            d_g3 = pltpu.async_copy(tab.at[idx3_v],
                                    gbuf.at[pl.ds(9 * L, 4 * L)], sem_c)

            d_cen.wait()
            d_g1.wait()
            d_g2.wait()
            pltpu.async_copy(cen_v, out.at[0, pl.ds(base, L)], sem_w)

            def max_body(d, _):
                for j in range(NB):
                    sl = pl.ds(j * L, L)
                    mx = gbuf[d, sl]
                    for m in range(1, 9):
                        mx = jnp.maximum(mx, gbuf[m * L + d, sl])
                    o2_v[d, sl] = mx
                return _

            lax.fori_loop(0, L, max_body, None)

            d_g3.wait()

            def bil_body(d, _):
                d_idx = jnp.broadcast_to(d, (L,))
                wb = [_lane_gather(w_v[k], d_idx) for k in range(4)]
                for j in range(NB):
                    sl = pl.ds(j * L, L)
                    acc = wb[0] * gbuf[9 * L + d, sl]
                    acc = acc + wb[1] * gbuf[10 * L + d, sl]
                    acc = acc + wb[2] * gbuf[11 * L + d, sl]
                    acc = acc + wb[3] * gbuf[12 * L + d, sl]
                    o1_v[d, sl] = acc
                return _

            lax.fori_loop(0, L, bil_body, None)
            pltpu.async_copy(o1_v, out.at[1, pl.ds(base, L)], sem_w)
            pltpu.async_copy(o2_v, out.at[2, pl.ds(base, L)], sem_w)

        return _

    lax.fori_loop(0, CPW, chunk_body, None)

    pltpu.make_async_copy(cen_v, out.at[0, pl.ds(0, L)], sem_w).wait()
    pltpu.make_async_copy(o1_v, out.at[1, pl.ds(0, L)], sem_w).wait()
    pltpu.make_async_copy(o2_v, out.at[2, pl.ds(0, L)], sem_w).wait()


@jax.jit
def kernel(params_maps, offset_maps, batch_ids, flat_inds):
    tab = jnp.transpose(params_maps, (0, 2, 3, 1)).reshape(B * HW, C)
    dyt = offset_maps[:, 0, :, :].reshape(B * HW)
    dxt = offset_maps[:, 1, :, :].reshape(B * HW)

    mesh = plsc.VectorSubcoreMesh(core_axis_name="c", subcore_axis_name="s")
    f = pl.kernel(
        _sc_body,
        mesh=mesh,
        out_type=jax.ShapeDtypeStruct((3, N, C), jnp.float32),
        scratch_types=[
            pltpu.VMEM((L,), jnp.int32),
            pltpu.VMEM((L,), jnp.int32),
            pltpu.VMEM((L,), jnp.float32),
            pltpu.VMEM((L,), jnp.float32),
            pltpu.VMEM((4, L), jnp.float32),
            pltpu.VMEM((8 * L,), jnp.int32),
            pltpu.VMEM((4 * L,), jnp.int32),
            pltpu.VMEM((L, C), jnp.float32),
            pltpu.VMEM((13 * L, C), jnp.float32),
            pltpu.VMEM((L, C), jnp.float32),
            pltpu.VMEM((L, C), jnp.float32),
            pltpu.SemaphoreType.DMA,
            pltpu.SemaphoreType.DMA,
            pltpu.SemaphoreType.DMA,
            pltpu.SemaphoreType.DMA,
        ],
    )
    return f(tab, dyt, dxt, batch_ids, flat_inds)
